# baseline (speedup 1.0000x reference)
import functools

import jax
import jax.numpy as jnp
from jax import lax
from jax.experimental import pallas as pl
from jax.experimental.pallas import tpu as pltpu
from jax.experimental.pallas import tpu_sc as plsc

VOCAB_DIM = 1000000
EMB = 64
SEQ = 200

_info = plsc.get_sparse_core_info()
_NC, _NS = _info.num_cores, _info.num_subcores
_NW = _NC * _NS
_B_PAD = 256
_ELEM_PER_W = _B_PAD * EMB // _NW
_CHUNKS = _ELEM_PER_W // 128


@functools.partial(
    pl.kernel,
    mesh=plsc.VectorSubcoreMesh(core_axis_name="c", subcore_axis_name="s"),
    out_type=jax.ShapeDtypeStruct((_B_PAD * EMB,), jnp.float32),
    scratch_types=[
        pltpu.VMEM((_CHUNKS, 128), jnp.int32),
        pltpu.VMEM((_ELEM_PER_W,), jnp.float32),
        pltpu.SemaphoreType.DMA,
    ],
)
def _sc_gather(table_hbm, idx_hbm, out_hbm, idx_v, vals_v, sem):
    wid = lax.axis_index("s") * _NC + lax.axis_index("c")
    pltpu.sync_copy(idx_hbm.at[wid], idx_v)
    copies = [
        pltpu.async_copy(
            table_hbm.at[idx_v.at[c]], vals_v.at[pl.ds(c * 128, 128)], sem
        )
        for c in range(_CHUNKS)
    ]
    for cp in copies:
        cp.wait()
    pltpu.sync_copy(vals_v, out_hbm.at[pl.ds(wid * _ELEM_PER_W, _ELEM_PER_W)])


@jax.jit
def kernel(x, emb_mat):
    idx = x.reshape(-1).astype(jnp.int32)
    idx = jnp.concatenate([idx, jnp.zeros((_B_PAD - SEQ,), jnp.int32)])
    elem_idx = (idx[:, None] * EMB + jnp.arange(EMB, dtype=jnp.int32)).reshape(
        _NW, _CHUNKS, 128
    )
    out = _sc_gather(emb_mat.reshape(-1), elem_idx)
    return out[: SEQ * EMB].reshape(1, SEQ, EMB)


# speedup vs baseline: 1.7827x; 1.7827x over previous
import functools

import jax
import jax.numpy as jnp
from jax import lax
from jax.experimental import pallas as pl
from jax.experimental.pallas import tpu as pltpu
from jax.experimental.pallas import tpu_sc as plsc

VOCAB_DIM = 1000000
EMB = 64
SEQ = 200

_info = plsc.get_sparse_core_info()
_NC, _NS = _info.num_cores, _info.num_subcores
_ROWS_PER_W = 8
_N_ACTIVE = SEQ // _ROWS_PER_W


@functools.partial(
    pl.kernel,
    mesh=plsc.VectorSubcoreMesh(core_axis_name="c", subcore_axis_name="s"),
    out_type=jax.ShapeDtypeStruct((SEQ, EMB), jnp.float32),
    scratch_types=[
        pltpu.VMEM((16,), jnp.int32),
        pltpu.VMEM((_ROWS_PER_W, EMB), jnp.float32),
        pltpu.SemaphoreType.DMA,
    ],
)
def _sc_gather(table_hbm, idx_hbm, out_hbm, idx_v, rows_v, sem):
    wid = lax.axis_index("s") * _NC + lax.axis_index("c")

    @pl.when(wid < _N_ACTIVE)
    def _():
        base = wid * _ROWS_PER_W
        pltpu.sync_copy(
            idx_hbm.at[pl.ds(base, _ROWS_PER_W)], idx_v.at[pl.ds(0, _ROWS_PER_W)]
        )
        idx_vec = idx_v[...]
        copies = []
        for j in range(_ROWS_PER_W):
            row = idx_vec[j]
            copies.append(
                pltpu.async_copy(
                    table_hbm.at[pl.ds(row, 1)], rows_v.at[pl.ds(j, 1)], sem
                )
            )
        for cp in copies:
            cp.wait()
        pltpu.sync_copy(rows_v, out_hbm.at[pl.ds(base, _ROWS_PER_W)])


@jax.jit
def kernel(x, emb_mat):
    idx = x.reshape(-1).astype(jnp.int32)
    out = _sc_gather(emb_mat, idx)
    return out.reshape(1, SEQ, EMB)
